# Initial kernel scaffold; baseline (speedup 1.0000x reference)
import jax
import jax.numpy as jnp
from jax.experimental import pallas as pl


def kernel(actions, embed_table):
    raise NotImplementedError("write your pallas kernel here")


# speedup vs baseline: 7.0634x; 7.0634x over previous
import functools

import jax
import jax.numpy as jnp
from jax import lax
from jax.experimental import pallas as pl
from jax.experimental.pallas import tpu as pltpu
from jax.experimental.pallas import tpu_sc as plsc

NC, NS, L = 2, 16, 16
NW = NC * NS
B = 4096
NT = 26
D = 128
NV = D // L
BPW = B // NW
BC = 16
NCHUNK = BPW // BC
ROWS = NT * BC

_mesh = plsc.VectorSubcoreMesh(core_axis_name="c", subcore_axis_name="s")


_scratch = [
    pltpu.VMEM((NT, BPW), jnp.int32),
    pltpu.VMEM((NT * BPW,), jnp.int32),
    pltpu.VMEM((ROWS, D), jnp.float32),
    pltpu.VMEM((BC, D), jnp.float32),
    pltpu.SemaphoreType.DMA,
]


def _embed_pool_body(aw_hbm, table_hbm, out_hbm, act_v, idx_v, gbuf, obuf, sem):
    wid = lax.axis_index("s") * NC + lax.axis_index("c")
    base = wid * BPW

    pltpu.sync_copy(aw_hbm.at[wid], act_v)

    for t in range(NT):
        for c in range(NCHUNK):
            idx_v[pl.ds(c * ROWS + t * BC, BC)] = act_v[t, pl.ds(c * BC, BC)] + t * 1000

    @pl.loop(0, NCHUNK)
    def _chunk(c):
        pltpu.async_copy(
            table_hbm.at[idx_v.at[pl.ds(c * ROWS, ROWS)]], gbuf, sem
        ).wait()
        for jj in range(BC):
            def body(t, accs):
                return tuple(
                    a + gbuf[t * BC + jj, pl.ds(v * L, L)]
                    for v, a in enumerate(accs)
                )
            accs = tuple(gbuf[jj, pl.ds(v * L, L)] for v in range(NV))
            accs = lax.fori_loop(1, NT, body, accs)
            for v in range(NV):
                obuf[jj, pl.ds(v * L, L)] = accs[v]
        pltpu.sync_copy(obuf, out_hbm.at[pl.ds(base + c * BC, BC)])


_embed_pool = pl.kernel(
    _embed_pool_body,
    out_type=jax.ShapeDtypeStruct((B, D), jnp.float32),
    mesh=_mesh,
    scratch_types=_scratch,
)


def kernel(actions, embed_table):
    aw = actions.astype(jnp.int32).T.reshape(NT, NW, BPW).transpose(1, 0, 2)
    return _embed_pool(aw, embed_table)


# speedup vs baseline: 7.8518x; 1.1116x over previous
import jax
import jax.numpy as jnp
from jax import lax
from jax.experimental import pallas as pl
from jax.experimental.pallas import tpu as pltpu
from jax.experimental.pallas import tpu_sc as plsc

NC, NS, L = 2, 16, 16
NW = NC * NS
B = 4096
NT = 26
D = 128
NV = D // L
BPW = B // NW
BC = 16
NCHUNK = BPW // BC
ROWS = NT * BC

_mesh = plsc.VectorSubcoreMesh(core_axis_name="c", subcore_axis_name="s")

_scratch = [
    pltpu.VMEM((NT, BPW), jnp.int32),
    pltpu.VMEM((NT * BPW,), jnp.int32),
    pltpu.VMEM((ROWS, D), jnp.float32),
    pltpu.VMEM((ROWS, D), jnp.float32),
    pltpu.VMEM((BC, D), jnp.float32),
    pltpu.SemaphoreType.DMA,
    pltpu.SemaphoreType.DMA,
]


def _embed_pool_body(aw_hbm, table_hbm, out_hbm,
                     act_v, idx_v, gbuf0, gbuf1, obuf, sem0, sem1):
    wid = lax.axis_index("s") * NC + lax.axis_index("c")
    base = wid * BPW

    pltpu.sync_copy(aw_hbm.at[wid], act_v)

    for t in range(NT):
        for c in range(NCHUNK):
            idx_v[pl.ds(c * ROWS + t * BC, BC)] = act_v[t, pl.ds(c * BC, BC)] + t * 1000

    bufs = ((gbuf0, sem0), (gbuf1, sem1))

    def start_gather(c, buf, sem):
        pltpu.async_copy(table_hbm.at[idx_v.at[pl.ds(c * ROWS, ROWS)]], buf, sem)

    start_gather(0, gbuf0, sem0)
    start_gather(1, gbuf1, sem1)

    @pl.loop(0, NCHUNK, step=2)
    def _pair(c0):
        for b in range(2):
            gbuf, sem = bufs[b]
            c = c0 + b
            pltpu.make_async_copy(
                table_hbm.at[idx_v.at[pl.ds(c * ROWS, ROWS)]], gbuf, sem
            ).wait()
            for jj in range(BC):
                def body(t, accs):
                    return tuple(
                        a + gbuf[t * BC + jj, pl.ds(v * L, L)]
                        for v, a in enumerate(accs)
                    )
                accs = tuple(gbuf[jj, pl.ds(v * L, L)] for v in range(NV))
                accs = lax.fori_loop(1, NT, body, accs, unroll=5)
                for v in range(NV):
                    obuf[jj, pl.ds(v * L, L)] = accs[v]

            @pl.when(c + 2 < NCHUNK)
            def _():
                start_gather(c + 2, gbuf, sem)

            pltpu.sync_copy(obuf, out_hbm.at[pl.ds(base + c * BC, BC)])


_embed_pool = pl.kernel(
    _embed_pool_body,
    out_type=jax.ShapeDtypeStruct((B, D), jnp.float32),
    mesh=_mesh,
    scratch_types=_scratch,
)


def kernel(actions, embed_table):
    aw = actions.astype(jnp.int32).T.reshape(NT, NW, BPW).transpose(1, 0, 2)
    return _embed_pool(aw, embed_table)
